# Initial kernel scaffold; baseline (speedup 1.0000x reference)
import jax
import jax.numpy as jnp
from jax.experimental import pallas as pl


def kernel(h, pos, edge_index, edge_attr, dist, time_emb, adj_extra, W_t, b_t, W_in, b_in, W1, b1, W2, coors_scale):
    raise NotImplementedError("write your pallas kernel here")


# speedup vs baseline: 4.6048x; 4.6048x over previous
import functools

import jax
import jax.numpy as jnp
from jax import lax
from jax.experimental import pallas as pl
from jax.experimental.pallas import tpu as pltpu
from jax.experimental.pallas import tpu_sc as plsc

N = 10000
E = 320000
HID = 128
NW = 32
EPW = E // NW
GK = 80
GITERS = EPW // GK
ACC_LEN = 30720
BE = 2000
NBLK = E // BE


def _prep(h, W_r, W_c, b_in2, interpret=False):
    def body(h_ref, wr_ref, wc_ref, b_ref, a_ref, b_out_ref):
        hb = h_ref[...]
        a_ref[...] = jnp.dot(hb, wr_ref[...],
                             preferred_element_type=jnp.float32) + b_ref[...]
        b_out_ref[...] = jnp.dot(hb, wc_ref[...],
                                 preferred_element_type=jnp.float32)

    grid = (5,)
    blk = N // 5
    return pl.pallas_call(
        body,
        grid=grid,
        in_specs=[
            pl.BlockSpec((blk, HID), lambda i: (i, 0)),
            pl.BlockSpec((HID, HID), lambda i: (0, 0)),
            pl.BlockSpec((HID, HID), lambda i: (0, 0)),
            pl.BlockSpec((1, HID), lambda i: (0, 0)),
        ],
        out_specs=[
            pl.BlockSpec((blk, HID), lambda i: (i, 0)),
            pl.BlockSpec((blk, HID), lambda i: (i, 0)),
        ],
        out_shape=[
            jax.ShapeDtypeStruct((N, HID), jnp.float32),
            jax.ShapeDtypeStruct((N, HID), jnp.float32),
        ],
        interpret=interpret,
    )(h, W_r, W_c, b_in2)


def _gather(A, B, row, col, interpret=False):
    info = plsc.get_sparse_core_info()
    nc = info.num_cores
    mesh = plsc.VectorSubcoreMesh(core_axis_name="c", subcore_axis_name="s")

    @functools.partial(
        pl.kernel,
        mesh=mesh,
        out_type=[
            jax.ShapeDtypeStruct((E, HID), jnp.float32),
            jax.ShapeDtypeStruct((E, HID), jnp.float32),
        ],
        scratch_types=[
            pltpu.VMEM((GK,), jnp.int32),
            pltpu.VMEM((GK,), jnp.int32),
            pltpu.VMEM((GK, HID), jnp.float32),
            pltpu.VMEM((GK, HID), jnp.float32),
            pltpu.SemaphoreType.DMA,
            pltpu.SemaphoreType.DMA,
        ],
        interpret=interpret,
    )
    def gk(a_hbm, b_hbm, row_hbm, col_hbm, ga_hbm, gb_hbm,
           idxr_v, idxc_v, bufa_v, bufb_v, sema, semb):
        wid = lax.axis_index("s") * nc + lax.axis_index("c")
        base = wid * EPW

        def body(j, _):
            off = base + j * GK
            pltpu.sync_copy(row_hbm.at[pl.ds(off, GK)], idxr_v)
            pltpu.sync_copy(col_hbm.at[pl.ds(off, GK)], idxc_v)
            cpa = pltpu.async_copy(a_hbm.at[idxr_v], bufa_v, sema)
            cpb = pltpu.async_copy(b_hbm.at[idxc_v], bufb_v, semb)
            cpa.wait()
            pltpu.sync_copy(bufa_v, ga_hbm.at[pl.ds(off, GK)])
            cpb.wait()
            pltpu.sync_copy(bufb_v, gb_hbm.at[pl.ds(off, GK)])
            return 0

        lax.fori_loop(0, GITERS, body, 0)

    return gk(A, B, row, col)


def _edge_mlp(GA, GB, edge_attr, dist, time_emb, adj_extra,
              Wa, Wd, Wts, Wtc, W1, small, interpret=False):
    def body(ga_ref, gb_ref, ea_ref, d_ref, te_ref, adj_ref,
             wa_ref, wd_ref, wts_ref, wtc_ref, w1_ref, sm_ref, s_ref):
        pre = (ga_ref[...] + gb_ref[...]
               + jnp.dot(ea_ref[...], wa_ref[...],
                         preferred_element_type=jnp.float32)
               + jnp.dot(d_ref[...], wd_ref[...],
                         preferred_element_type=jnp.float32))
        mu = jnp.mean(pre, axis=-1, keepdims=True)
        xc = pre - mu
        var = jnp.mean(xc * xc, axis=-1, keepdims=True)
        ln = xc / jnp.sqrt(var + 1e-6)

        te = te_ref[...]
        st = te / (1.0 + jnp.exp(-te))
        sm = sm_ref[...]
        shift = jnp.dot(st, wts_ref[...],
                        preferred_element_type=jnp.float32) + sm[0:1, :]
        scale = jnp.dot(st, wtc_ref[...],
                        preferred_element_type=jnp.float32) + sm[1:2, :]
        inv = ln * (1.0 + scale) + shift

        u = jnp.dot(inv, w1_ref[...],
                    preferred_element_type=jnp.float32) + sm[2:3, :]
        u = u / (1.0 + jnp.exp(-u))
        v0 = jnp.sum(u * sm[3:4, :], axis=-1, keepdims=True)
        v1 = jnp.sum(u * sm[4:5, :], axis=-1, keepdims=True)
        v2 = jnp.sum(u * sm[5:6, :], axis=-1, keepdims=True)
        adj = adj_ref[...]
        s = (jnp.tanh(v0)
             + jnp.tanh(v1) * adj[:, 0:1]
             + jnp.tanh(v2) * adj[:, 1:2]) * sm_ref[6, 0]
        s_ref[...] = s

    return pl.pallas_call(
        body,
        grid=(NBLK,),
        in_specs=[
            pl.BlockSpec((BE, HID), lambda i: (i, 0)),
            pl.BlockSpec((BE, HID), lambda i: (i, 0)),
            pl.BlockSpec((BE, 16), lambda i: (i, 0)),
            pl.BlockSpec((BE, 16), lambda i: (i, 0)),
            pl.BlockSpec((BE, HID), lambda i: (i, 0)),
            pl.BlockSpec((BE, 2), lambda i: (i, 0)),
            pl.BlockSpec((16, HID), lambda i: (0, 0)),
            pl.BlockSpec((16, HID), lambda i: (0, 0)),
            pl.BlockSpec((HID, HID), lambda i: (0, 0)),
            pl.BlockSpec((HID, HID), lambda i: (0, 0)),
            pl.BlockSpec((HID, HID), lambda i: (0, 0)),
            pl.BlockSpec((8, HID), lambda i: (0, 0)),
        ],
        out_specs=pl.BlockSpec((BE, 1), lambda i: (i, 0)),
        out_shape=jax.ShapeDtypeStruct((E, 1), jnp.float32),
        interpret=interpret,
    )(GA, GB, edge_attr, dist, time_emb, adj_extra,
      Wa, Wd, Wts, Wtc, W1, small)


def _scatter(pos_flat, row, col, s, interpret=False):
    info = plsc.get_sparse_core_info()
    nc = info.num_cores
    mesh = plsc.VectorSubcoreMesh(core_axis_name="c", subcore_axis_name="s")
    groups = EPW // 16

    @functools.partial(
        pl.kernel,
        mesh=mesh,
        out_type=jax.ShapeDtypeStruct((NW, ACC_LEN), jnp.float32),
        scratch_types=[
            pltpu.VMEM((3 * N,), jnp.float32),
            pltpu.VMEM((ACC_LEN,), jnp.float32),
            pltpu.VMEM((EPW,), jnp.int32),
            pltpu.VMEM((EPW,), jnp.int32),
            pltpu.VMEM((EPW,), jnp.float32),
        ],
        compiler_params=pltpu.CompilerParams(needs_layout_passes=False),
        interpret=interpret,
    )
    def sk(pos_hbm, row_hbm, col_hbm, s_hbm, out_hbm,
           pos_v, acc_v, rv, cv, sv):
        wid = lax.axis_index("s") * nc + lax.axis_index("c")
        base = wid * EPW
        pltpu.sync_copy(pos_hbm, pos_v)
        pltpu.sync_copy(row_hbm.at[pl.ds(base, EPW)], rv)
        pltpu.sync_copy(col_hbm.at[pl.ds(base, EPW)], cv)
        pltpu.sync_copy(s_hbm.at[pl.ds(base, EPW)], sv)

        zeros16 = jnp.zeros((16,), jnp.float32)

        def zbody(i, _):
            acc_v[pl.ds(i * 16, 16)] = zeros16
            return 0

        lax.fori_loop(0, ACC_LEN // 16, zbody, 0)

        def ebody(g, _):
            o = g * 16
            r3 = rv[pl.ds(o, 16)] * 3
            c3 = cv[pl.ds(o, 16)] * 3
            se = sv[pl.ds(o, 16)]
            dx = (plsc.load_gather(pos_v, [r3])
                  - plsc.load_gather(pos_v, [c3]))
            dy = (plsc.load_gather(pos_v, [r3 + 1])
                  - plsc.load_gather(pos_v, [c3 + 1]))
            dz = (plsc.load_gather(pos_v, [r3 + 2])
                  - plsc.load_gather(pos_v, [c3 + 2]))
            n2 = jnp.maximum(dx * dx + dy * dy + dz * dz, 1e-30)
            y = plsc.bitcast(0x5F3759DF - (plsc.bitcast(n2, jnp.int32) >> 1),
                             jnp.float32)
            y = y * (1.5 - 0.5 * n2 * y * y)
            y = y * (1.5 - 0.5 * n2 * y * y)
            y = y * (1.5 - 0.5 * n2 * y * y)
            nrm = n2 * y
            f = se / jnp.maximum(nrm, 1e-8)
            plsc.addupdate_scatter(acc_v, [r3], dx * f)
            plsc.addupdate_scatter(acc_v, [r3 + 1], dy * f)
            plsc.addupdate_scatter(acc_v, [r3 + 2], dz * f)
            return 0

        lax.fori_loop(0, groups, ebody, 0)
        pltpu.sync_copy(acc_v, out_hbm.at[wid])

    return sk(pos_flat, row, col, s)


def _combine(partials3, pos_pad2, interpret=False):
    def body(p_ref, pos_ref, o_ref):
        o_ref[...] = jnp.sum(p_ref[...], axis=0) + pos_ref[...]

    return pl.pallas_call(
        body,
        grid=(1,),
        in_specs=[
            pl.BlockSpec((NW, 240, 128), lambda i: (0, 0, 0)),
            pl.BlockSpec((240, 128), lambda i: (0, 0)),
        ],
        out_specs=pl.BlockSpec((240, 128), lambda i: (0, 0)),
        out_shape=jax.ShapeDtypeStruct((240, 128), jnp.float32),
        interpret=interpret,
    )(partials3, pos_pad2)


def kernel(h, pos, edge_index, edge_attr, dist, time_emb, adj_extra,
           W_t, b_t, W_in, b_in, W1, b1, W2, coors_scale):
    row = edge_index[0]
    col = edge_index[1]
    W_r = W_in[:HID]
    W_c = W_in[HID:2 * HID]
    Wa = W_in[2 * HID:2 * HID + 16]
    Wd = W_in[2 * HID + 16:]
    Wts = W_t[:, :HID]
    Wtc = W_t[:, HID:]

    small = jnp.zeros((8, HID), jnp.float32)
    small = small.at[0].set(b_t[:HID]).at[1].set(b_t[HID:]).at[2].set(b1)
    small = small.at[3].set(W2[:, 0]).at[4].set(W2[:, 1]).at[5].set(W2[:, 2])
    small = small.at[6, 0].set(coors_scale / 3.0)

    A, B = _prep(h, W_r, W_c, b_in.reshape(1, HID))
    GA, GB = _gather(A, B, row, col)
    s = _edge_mlp(GA, GB, edge_attr, dist, time_emb, adj_extra,
                  Wa, Wd, Wts, Wtc, W1, small)
    partials = _scatter(pos.reshape(-1), row, col, s.reshape(E))
    pos_pad2 = jnp.pad(pos.reshape(-1), (0, ACC_LEN - 3 * N)).reshape(240, 128)
    out = _combine(partials.reshape(NW, 240, 128), pos_pad2)
    return out.reshape(ACC_LEN)[:3 * N].reshape(N, 3)


# speedup vs baseline: 4.7622x; 1.0342x over previous
import functools

import jax
import jax.numpy as jnp
from jax import lax
from jax.experimental import pallas as pl
from jax.experimental.pallas import tpu as pltpu
from jax.experimental.pallas import tpu_sc as plsc

N = 10000
E = 320000
HID = 128
NW = 32
EPW = E // NW
GK = 40
GITERS = EPW // GK
NBUF = 5
ACC_LEN = 30720
BE = 2000
NBLK = E // BE


def _prep(h, W_r, W_c, b_in2, interpret=False):
    def body(h_ref, wr_ref, wc_ref, b_ref, a_ref, b_out_ref):
        hb = h_ref[...]
        a_ref[...] = jnp.dot(hb, wr_ref[...],
                             preferred_element_type=jnp.float32) + b_ref[...]
        b_out_ref[...] = jnp.dot(hb, wc_ref[...],
                                 preferred_element_type=jnp.float32)

    grid = (5,)
    blk = N // 5
    return pl.pallas_call(
        body,
        grid=grid,
        in_specs=[
            pl.BlockSpec((blk, HID), lambda i: (i, 0)),
            pl.BlockSpec((HID, HID), lambda i: (0, 0)),
            pl.BlockSpec((HID, HID), lambda i: (0, 0)),
            pl.BlockSpec((1, HID), lambda i: (0, 0)),
        ],
        out_specs=[
            pl.BlockSpec((blk, HID), lambda i: (i, 0)),
            pl.BlockSpec((blk, HID), lambda i: (i, 0)),
        ],
        out_shape=[
            jax.ShapeDtypeStruct((N, HID), jnp.float32),
            jax.ShapeDtypeStruct((N, HID), jnp.float32),
        ],
        interpret=interpret,
    )(h, W_r, W_c, b_in2)


def _gather(A, B, row, col, interpret=False):
    info = plsc.get_sparse_core_info()
    nc = info.num_cores
    mesh = plsc.VectorSubcoreMesh(core_axis_name="c", subcore_axis_name="s")

    @functools.partial(
        pl.kernel,
        mesh=mesh,
        out_type=[
            jax.ShapeDtypeStruct((E, HID), jnp.float32),
            jax.ShapeDtypeStruct((E, HID), jnp.float32),
        ],
        scratch_types=(
            [pltpu.VMEM((EPW,), jnp.int32),
             pltpu.VMEM((EPW,), jnp.int32)]
            + [pltpu.VMEM((GK, HID), jnp.float32) for _ in range(2 * NBUF)]
            + [pltpu.SemaphoreType.DMA for _ in range(2 * NBUF)]
        ),
        interpret=interpret,
    )
    def gk(a_hbm, b_hbm, row_hbm, col_hbm, ga_hbm, gb_hbm, rv, cv, *rest):
        bufa = rest[0:NBUF]
        bufb = rest[NBUF:2 * NBUF]
        semg = rest[2 * NBUF:3 * NBUF]
        semw = rest[3 * NBUF:4 * NBUF]
        wid = lax.axis_index("s") * nc + lax.axis_index("c")
        base = wid * EPW
        pltpu.sync_copy(row_hbm.at[pl.ds(base, EPW)], rv)
        pltpu.sync_copy(col_hbm.at[pl.ds(base, EPW)], cv)

        def fire_gather(j, p):
            o = j * GK
            pltpu.make_async_copy(
                a_hbm.at[rv.at[pl.ds(o, GK)]], bufa[p], semg[p]).start()
            pltpu.make_async_copy(
                b_hbm.at[cv.at[pl.ds(o, GK)]], bufb[p], semg[p]).start()

        def drain_gather(p):
            pltpu.make_async_copy(a_hbm.at[rv.at[pl.ds(0, GK)]],
                                  bufa[p], semg[p]).wait()
            pltpu.make_async_copy(b_hbm.at[cv.at[pl.ds(0, GK)]],
                                  bufb[p], semg[p]).wait()

        def fire_write(j, p):
            off = base + j * GK
            pltpu.make_async_copy(
                bufa[p], ga_hbm.at[pl.ds(off, GK)], semw[p]).start()
            pltpu.make_async_copy(
                bufb[p], gb_hbm.at[pl.ds(off, GK)], semw[p]).start()

        def drain_write(p):
            pltpu.make_async_copy(bufa[p], ga_hbm.at[pl.ds(base, GK)],
                                  semw[p]).wait()
            pltpu.make_async_copy(bufb[p], gb_hbm.at[pl.ds(base, GK)],
                                  semw[p]).wait()

        fire_gather(0, 0)

        def group(g, _):
            for p in range(NBUF):
                j = g * NBUF + p
                nxt = (p + 1) % NBUF

                @pl.when(j + 1 < GITERS)
                def _():
                    @pl.when(j + 1 >= NBUF)
                    def _():
                        drain_write(nxt)
                    fire_gather(j + 1, nxt)

                drain_gather(p)
                fire_write(j, p)
            return 0

        lax.fori_loop(0, GITERS // NBUF, group, 0)
        for p in range(NBUF):
            drain_write(p)

    return gk(A, B, row, col)


def _edge_mlp(GA, GB, edge_attr, dist, time_emb, adj_extra,
              Wa, Wd, Wts, Wtc, W1, small, interpret=False):
    def body(ga_ref, gb_ref, ea_ref, d_ref, te_ref, adj_ref,
             wa_ref, wd_ref, wts_ref, wtc_ref, w1_ref, sm_ref, s_ref):
        pre = (ga_ref[...] + gb_ref[...]
               + jnp.dot(ea_ref[...], wa_ref[...],
                         preferred_element_type=jnp.float32)
               + jnp.dot(d_ref[...], wd_ref[...],
                         preferred_element_type=jnp.float32))
        mu = jnp.mean(pre, axis=-1, keepdims=True)
        xc = pre - mu
        var = jnp.mean(xc * xc, axis=-1, keepdims=True)
        ln = xc / jnp.sqrt(var + 1e-6)

        te = te_ref[...]
        st = te / (1.0 + jnp.exp(-te))
        sm = sm_ref[...]
        shift = jnp.dot(st, wts_ref[...],
                        preferred_element_type=jnp.float32) + sm[0:1, :]
        scale = jnp.dot(st, wtc_ref[...],
                        preferred_element_type=jnp.float32) + sm[1:2, :]
        inv = ln * (1.0 + scale) + shift

        u = jnp.dot(inv, w1_ref[...],
                    preferred_element_type=jnp.float32) + sm[2:3, :]
        u = u / (1.0 + jnp.exp(-u))
        v0 = jnp.sum(u * sm[3:4, :], axis=-1, keepdims=True)
        v1 = jnp.sum(u * sm[4:5, :], axis=-1, keepdims=True)
        v2 = jnp.sum(u * sm[5:6, :], axis=-1, keepdims=True)
        adj = adj_ref[...]
        s = (jnp.tanh(v0)
             + jnp.tanh(v1) * adj[:, 0:1]
             + jnp.tanh(v2) * adj[:, 1:2]) * sm_ref[6, 0]
        s_ref[...] = s

    return pl.pallas_call(
        body,
        grid=(NBLK,),
        in_specs=[
            pl.BlockSpec((BE, HID), lambda i: (i, 0)),
            pl.BlockSpec((BE, HID), lambda i: (i, 0)),
            pl.BlockSpec((BE, 16), lambda i: (i, 0)),
            pl.BlockSpec((BE, 16), lambda i: (i, 0)),
            pl.BlockSpec((BE, HID), lambda i: (i, 0)),
            pl.BlockSpec((BE, 2), lambda i: (i, 0)),
            pl.BlockSpec((16, HID), lambda i: (0, 0)),
            pl.BlockSpec((16, HID), lambda i: (0, 0)),
            pl.BlockSpec((HID, HID), lambda i: (0, 0)),
            pl.BlockSpec((HID, HID), lambda i: (0, 0)),
            pl.BlockSpec((HID, HID), lambda i: (0, 0)),
            pl.BlockSpec((8, HID), lambda i: (0, 0)),
        ],
        out_specs=pl.BlockSpec((BE, 1), lambda i: (i, 0)),
        out_shape=jax.ShapeDtypeStruct((E, 1), jnp.float32),
        interpret=interpret,
    )(GA, GB, edge_attr, dist, time_emb, adj_extra,
      Wa, Wd, Wts, Wtc, W1, small)


def _scatter(pos_flat, row, col, s, interpret=False):
    info = plsc.get_sparse_core_info()
    nc = info.num_cores
    mesh = plsc.VectorSubcoreMesh(core_axis_name="c", subcore_axis_name="s")
    groups = EPW // 16

    @functools.partial(
        pl.kernel,
        mesh=mesh,
        out_type=jax.ShapeDtypeStruct((NW, ACC_LEN), jnp.float32),
        scratch_types=[
            pltpu.VMEM((3 * N,), jnp.float32),
            pltpu.VMEM((ACC_LEN,), jnp.float32),
            pltpu.VMEM((EPW,), jnp.int32),
            pltpu.VMEM((EPW,), jnp.int32),
            pltpu.VMEM((EPW,), jnp.float32),
        ],
        compiler_params=pltpu.CompilerParams(needs_layout_passes=False),
        interpret=interpret,
    )
    def sk(pos_hbm, row_hbm, col_hbm, s_hbm, out_hbm,
           pos_v, acc_v, rv, cv, sv):
        wid = lax.axis_index("s") * nc + lax.axis_index("c")
        base = wid * EPW
        pltpu.sync_copy(pos_hbm, pos_v)
        pltpu.sync_copy(row_hbm.at[pl.ds(base, EPW)], rv)
        pltpu.sync_copy(col_hbm.at[pl.ds(base, EPW)], cv)
        pltpu.sync_copy(s_hbm.at[pl.ds(base, EPW)], sv)

        zeros16 = jnp.zeros((16,), jnp.float32)

        def zbody(i, _):
            acc_v[pl.ds(i * 16, 16)] = zeros16
            return 0

        lax.fori_loop(0, ACC_LEN // 16, zbody, 0)

        def ebody(g, _):
            o = g * 16
            r3 = rv[pl.ds(o, 16)] * 3
            c3 = cv[pl.ds(o, 16)] * 3
            se = sv[pl.ds(o, 16)]
            dx = (plsc.load_gather(pos_v, [r3])
                  - plsc.load_gather(pos_v, [c3]))
            dy = (plsc.load_gather(pos_v, [r3 + 1])
                  - plsc.load_gather(pos_v, [c3 + 1]))
            dz = (plsc.load_gather(pos_v, [r3 + 2])
                  - plsc.load_gather(pos_v, [c3 + 2]))
            n2 = jnp.maximum(dx * dx + dy * dy + dz * dz, 1e-30)
            y = plsc.bitcast(0x5F3759DF - (plsc.bitcast(n2, jnp.int32) >> 1),
                             jnp.float32)
            y = y * (1.5 - 0.5 * n2 * y * y)
            y = y * (1.5 - 0.5 * n2 * y * y)
            y = y * (1.5 - 0.5 * n2 * y * y)
            nrm = n2 * y
            f = se / jnp.maximum(nrm, 1e-8)
            plsc.addupdate_scatter(acc_v, [r3], dx * f)
            plsc.addupdate_scatter(acc_v, [r3 + 1], dy * f)
            plsc.addupdate_scatter(acc_v, [r3 + 2], dz * f)
            return 0

        lax.fori_loop(0, groups, ebody, 0)
        pltpu.sync_copy(acc_v, out_hbm.at[wid])

    return sk(pos_flat, row, col, s)


def _combine(partials3, pos_pad2, interpret=False):
    def body(p_ref, pos_ref, o_ref):
        o_ref[...] = jnp.sum(p_ref[...], axis=0) + pos_ref[...]

    return pl.pallas_call(
        body,
        grid=(1,),
        in_specs=[
            pl.BlockSpec((NW, 240, 128), lambda i: (0, 0, 0)),
            pl.BlockSpec((240, 128), lambda i: (0, 0)),
        ],
        out_specs=pl.BlockSpec((240, 128), lambda i: (0, 0)),
        out_shape=jax.ShapeDtypeStruct((240, 128), jnp.float32),
        interpret=interpret,
    )(partials3, pos_pad2)


def kernel(h, pos, edge_index, edge_attr, dist, time_emb, adj_extra,
           W_t, b_t, W_in, b_in, W1, b1, W2, coors_scale):
    row = edge_index[0]
    col = edge_index[1]
    W_r = W_in[:HID]
    W_c = W_in[HID:2 * HID]
    Wa = W_in[2 * HID:2 * HID + 16]
    Wd = W_in[2 * HID + 16:]
    Wts = W_t[:, :HID]
    Wtc = W_t[:, HID:]

    small = jnp.zeros((8, HID), jnp.float32)
    small = small.at[0].set(b_t[:HID]).at[1].set(b_t[HID:]).at[2].set(b1)
    small = small.at[3].set(W2[:, 0]).at[4].set(W2[:, 1]).at[5].set(W2[:, 2])
    small = small.at[6, 0].set(coors_scale / 3.0)

    A, B = _prep(h, W_r, W_c, b_in.reshape(1, HID))
    GA, GB = _gather(A, B, row, col)
    s = _edge_mlp(GA, GB, edge_attr, dist, time_emb, adj_extra,
                  Wa, Wd, Wts, Wtc, W1, small)
    partials = _scatter(pos.reshape(-1), row, col, s.reshape(E))
    pos_pad2 = jnp.pad(pos.reshape(-1), (0, ACC_LEN - 3 * N)).reshape(240, 128)
    out = _combine(partials.reshape(NW, 240, 128), pos_pad2)
    return out.reshape(ACC_LEN)[:3 * N].reshape(N, 3)


# speedup vs baseline: 5.6073x; 1.1775x over previous
import functools

import jax
import jax.numpy as jnp
from jax import lax
from jax.experimental import pallas as pl
from jax.experimental.pallas import tpu as pltpu
from jax.experimental.pallas import tpu_sc as plsc

N = 10000
E = 320000
HID = 128
NW = 32
GK = 40
NBUF = 5
ACC_LEN = 30720
BE = 2560
SLABS = (166400, 153600)


def _prep(h, W_r, W_c, b_in2, interpret=False):
    def body(h_ref, wr_ref, wc_ref, b_ref, a_ref, b_out_ref):
        hb = h_ref[...]
        a_ref[...] = jnp.dot(hb, wr_ref[...],
                             preferred_element_type=jnp.float32) + b_ref[...]
        b_out_ref[...] = jnp.dot(hb, wc_ref[...],
                                 preferred_element_type=jnp.float32)

    grid = (5,)
    blk = N // 5
    return pl.pallas_call(
        body,
        grid=grid,
        in_specs=[
            pl.BlockSpec((blk, HID), lambda i: (i, 0)),
            pl.BlockSpec((HID, HID), lambda i: (0, 0)),
            pl.BlockSpec((HID, HID), lambda i: (0, 0)),
            pl.BlockSpec((1, HID), lambda i: (0, 0)),
        ],
        out_specs=[
            pl.BlockSpec((blk, HID), lambda i: (i, 0)),
            pl.BlockSpec((blk, HID), lambda i: (i, 0)),
        ],
        out_shape=[
            jax.ShapeDtypeStruct((N, HID), jnp.float32),
            jax.ShapeDtypeStruct((N, HID), jnp.float32),
        ],
        interpret=interpret,
    )(h, W_r, W_c, b_in2)


def _gather(A, B, row, col, e0, ne, interpret=False):
    info = plsc.get_sparse_core_info()
    nc = info.num_cores
    mesh = plsc.VectorSubcoreMesh(core_axis_name="c", subcore_axis_name="s")
    epw = ne // NW
    giters = epw // GK

    @functools.partial(
        pl.kernel,
        mesh=mesh,
        out_type=jax.ShapeDtypeStruct((ne, HID), jnp.float32),
        scratch_types=(
            [pltpu.VMEM((epw,), jnp.int32),
             pltpu.VMEM((epw,), jnp.int32)]
            + [pltpu.VMEM((GK, HID), jnp.float32) for _ in range(2 * NBUF)]
            + [pltpu.SemaphoreType.DMA for _ in range(2 * NBUF)]
        ),
        interpret=interpret,
    )
    def gk(a_hbm, b_hbm, row_hbm, col_hbm, gab_hbm, rv, cv, *rest):
        bufa = rest[0:NBUF]
        bufb = rest[NBUF:2 * NBUF]
        semg = rest[2 * NBUF:3 * NBUF]
        semw = rest[3 * NBUF:4 * NBUF]
        wid = lax.axis_index("s") * nc + lax.axis_index("c")
        base = wid * epw
        pltpu.sync_copy(row_hbm.at[pl.ds(e0 + base, epw)], rv)
        pltpu.sync_copy(col_hbm.at[pl.ds(e0 + base, epw)], cv)

        def fire_gather(j, p):
            o = j * GK
            pltpu.make_async_copy(
                a_hbm.at[rv.at[pl.ds(o, GK)]], bufa[p], semg[p]).start()
            pltpu.make_async_copy(
                b_hbm.at[cv.at[pl.ds(o, GK)]], bufb[p], semg[p]).start()

        def drain_gather(p):
            pltpu.make_async_copy(a_hbm.at[rv.at[pl.ds(0, GK)]],
                                  bufa[p], semg[p]).wait()
            pltpu.make_async_copy(b_hbm.at[cv.at[pl.ds(0, GK)]],
                                  bufb[p], semg[p]).wait()

        def add_into_a(p):
            def rowbody(i, _):
                for c in range(HID // 16):
                    sl = pl.ds(c * 16, 16)
                    bufa[p][i, sl] = bufa[p][i, sl] + bufb[p][i, sl]
                return 0

            lax.fori_loop(0, GK, rowbody, 0)

        def fire_write(j, p):
            off = base + j * GK
            pltpu.make_async_copy(
                bufa[p], gab_hbm.at[pl.ds(off, GK)], semw[p]).start()

        def drain_write(p):
            pltpu.make_async_copy(bufa[p], gab_hbm.at[pl.ds(base, GK)],
                                  semw[p]).wait()

        fire_gather(0, 0)

        def group(g, _):
            for p in range(NBUF):
                j = g * NBUF + p
                nxt = (p + 1) % NBUF

                @pl.when(j + 1 < giters)
                def _():
                    @pl.when(j + 1 >= NBUF)
                    def _():
                        drain_write(nxt)
                    fire_gather(j + 1, nxt)

                drain_gather(p)
                add_into_a(p)
                fire_write(j, p)
            return 0

        lax.fori_loop(0, giters // NBUF, group, 0)
        for p in range(NBUF):
            drain_write(p)

    return gk(A, B, row, col)


def _edge_mlp(GAB, ea_t, d_t, time_emb, adj_t,
              Wa, Wd, Wts, Wtc, W1, small, interpret=False):
    tdims = (((0,), (0,)), ((), ()))

    def body(gab_ref, ea_ref, d_ref, te_ref, adj_ref,
             wa_ref, wd_ref, wts_ref, wtc_ref, w1_ref, sm_ref, s_ref):
        pre = (gab_ref[...]
               + lax.dot_general(ea_ref[...].astype(jnp.bfloat16),
                                 wa_ref[...], tdims,
                                 preferred_element_type=jnp.float32)
               + lax.dot_general(d_ref[...].astype(jnp.bfloat16),
                                 wd_ref[...], tdims,
                                 preferred_element_type=jnp.float32))
        mu = jnp.mean(pre, axis=-1, keepdims=True)
        xc = pre - mu
        var = jnp.mean(xc * xc, axis=-1, keepdims=True)
        ln = xc / jnp.sqrt(var + 1e-6)

        te = te_ref[...]
        st = (te / (1.0 + jnp.exp(-te))).astype(jnp.bfloat16)
        sm = sm_ref[...]
        shift = jnp.dot(st, wts_ref[...],
                        preferred_element_type=jnp.float32) + sm[0:1, :]
        scale = jnp.dot(st, wtc_ref[...],
                        preferred_element_type=jnp.float32) + sm[1:2, :]
        inv = (ln * (1.0 + scale) + shift).astype(jnp.bfloat16)

        u = jnp.dot(inv, w1_ref[...],
                    preferred_element_type=jnp.float32) + sm[2:3, :]
        u = u / (1.0 + jnp.exp(-u))
        v0 = jnp.sum(u * sm[3:4, :], axis=-1)
        v1 = jnp.sum(u * sm[4:5, :], axis=-1)
        v2 = jnp.sum(u * sm[5:6, :], axis=-1)
        adj = adj_ref[...]
        s = (jnp.tanh(v0)
             + jnp.tanh(v1) * adj[0, :]
             + jnp.tanh(v2) * adj[1, :]) * sm_ref[6, 0]
        s_ref[...] = s[None, None, :]

    ne = GAB.shape[0]
    return pl.pallas_call(
        body,
        grid=(ne // BE,),
        in_specs=[
            pl.BlockSpec((BE, HID), lambda i: (i, 0)),
            pl.BlockSpec((16, BE), lambda i: (0, i)),
            pl.BlockSpec((16, BE), lambda i: (0, i)),
            pl.BlockSpec((BE, HID), lambda i: (i, 0)),
            pl.BlockSpec((8, BE), lambda i: (0, i)),
            pl.BlockSpec((16, HID), lambda i: (0, 0)),
            pl.BlockSpec((16, HID), lambda i: (0, 0)),
            pl.BlockSpec((HID, HID), lambda i: (0, 0)),
            pl.BlockSpec((HID, HID), lambda i: (0, 0)),
            pl.BlockSpec((HID, HID), lambda i: (0, 0)),
            pl.BlockSpec((8, HID), lambda i: (0, 0)),
        ],
        out_specs=pl.BlockSpec((1, 1, BE), lambda i: (i, 0, 0)),
        out_shape=jax.ShapeDtypeStruct((ne // BE, 1, BE), jnp.float32),
        interpret=interpret,
    )(GAB, ea_t, d_t, time_emb, adj_t,
      Wa, Wd, Wts, Wtc, W1, small)


def _scatter(pos_flat, row, col, s, e0, ne, interpret=False):
    info = plsc.get_sparse_core_info()
    nc = info.num_cores
    mesh = plsc.VectorSubcoreMesh(core_axis_name="c", subcore_axis_name="s")
    epw = ne // NW
    groups = epw // 16

    @functools.partial(
        pl.kernel,
        mesh=mesh,
        out_type=jax.ShapeDtypeStruct((NW, ACC_LEN), jnp.float32),
        scratch_types=[
            pltpu.VMEM((3 * N,), jnp.float32),
            pltpu.VMEM((ACC_LEN,), jnp.float32),
            pltpu.VMEM((epw,), jnp.int32),
            pltpu.VMEM((epw,), jnp.int32),
            pltpu.VMEM((epw,), jnp.float32),
        ],
        compiler_params=pltpu.CompilerParams(needs_layout_passes=False),
        interpret=interpret,
    )
    def sk(pos_hbm, row_hbm, col_hbm, s_hbm, out_hbm,
           pos_v, acc_v, rv, cv, sv):
        wid = lax.axis_index("s") * nc + lax.axis_index("c")
        base = wid * epw
        pltpu.sync_copy(pos_hbm, pos_v)
        pltpu.sync_copy(row_hbm.at[pl.ds(e0 + base, epw)], rv)
        pltpu.sync_copy(col_hbm.at[pl.ds(e0 + base, epw)], cv)
        pltpu.sync_copy(s_hbm.at[pl.ds(base, epw)], sv)

        zeros16 = jnp.zeros((16,), jnp.float32)

        def zbody(i, _):
            acc_v[pl.ds(i * 16, 16)] = zeros16
            return 0

        lax.fori_loop(0, ACC_LEN // 16, zbody, 0)

        def ebody(g, _):
            o = g * 16
            r3 = rv[pl.ds(o, 16)] * 3
            c3 = cv[pl.ds(o, 16)] * 3
            se = sv[pl.ds(o, 16)]
            dx = (plsc.load_gather(pos_v, [r3])
                  - plsc.load_gather(pos_v, [c3]))
            dy = (plsc.load_gather(pos_v, [r3 + 1])
                  - plsc.load_gather(pos_v, [c3 + 1]))
            dz = (plsc.load_gather(pos_v, [r3 + 2])
                  - plsc.load_gather(pos_v, [c3 + 2]))
            n2 = jnp.maximum(dx * dx + dy * dy + dz * dz, 1e-30)
            y = plsc.bitcast(0x5F3759DF - (plsc.bitcast(n2, jnp.int32) >> 1),
                             jnp.float32)
            y = y * (1.5 - 0.5 * n2 * y * y)
            y = y * (1.5 - 0.5 * n2 * y * y)
            y = y * (1.5 - 0.5 * n2 * y * y)
            nrm = n2 * y
            f = se / jnp.maximum(nrm, 1e-8)
            plsc.addupdate_scatter(acc_v, [r3], dx * f)
            plsc.addupdate_scatter(acc_v, [r3 + 1], dy * f)
            plsc.addupdate_scatter(acc_v, [r3 + 2], dz * f)
            return 0

        lax.fori_loop(0, groups, ebody, 0)
        pltpu.sync_copy(acc_v, out_hbm.at[wid])

    return sk(pos_flat, row, col, s)


def _combine(p1, p2, pos_pad2, interpret=False):
    def body(p1_ref, p2_ref, pos_ref, o_ref):
        o_ref[...] = (jnp.sum(p1_ref[...], axis=0)
                      + jnp.sum(p2_ref[...], axis=0) + pos_ref[...])

    return pl.pallas_call(
        body,
        grid=(1,),
        in_specs=[
            pl.BlockSpec((NW, 240, 128), lambda i: (0, 0, 0)),
            pl.BlockSpec((NW, 240, 128), lambda i: (0, 0, 0)),
            pl.BlockSpec((240, 128), lambda i: (0, 0)),
        ],
        out_specs=pl.BlockSpec((240, 128), lambda i: (0, 0)),
        out_shape=jax.ShapeDtypeStruct((240, 128), jnp.float32),
        interpret=interpret,
    )(p1, p2, pos_pad2)


def kernel(h, pos, edge_index, edge_attr, dist, time_emb, adj_extra,
           W_t, b_t, W_in, b_in, W1, b1, W2, coors_scale):
    row = edge_index[0]
    col = edge_index[1]
    W_r = W_in[:HID]
    W_c = W_in[HID:2 * HID]
    Wa = W_in[2 * HID:2 * HID + 16].astype(jnp.bfloat16)
    Wd = W_in[2 * HID + 16:].astype(jnp.bfloat16)
    Wts = W_t[:, :HID].astype(jnp.bfloat16)
    Wtc = W_t[:, HID:].astype(jnp.bfloat16)
    W1b = W1.astype(jnp.bfloat16)

    small = jnp.zeros((8, HID), jnp.float32)
    small = small.at[0].set(b_t[:HID]).at[1].set(b_t[HID:]).at[2].set(b1)
    small = small.at[3].set(W2[:, 0]).at[4].set(W2[:, 1]).at[5].set(W2[:, 2])
    small = small.at[6, 0].set(coors_scale / 3.0)

    A, B = _prep(h, W_r, W_c, b_in.reshape(1, HID))
    pos_flat = pos.reshape(-1)
    ea_t = edge_attr.T
    d_t = dist.T
    adj_t = jnp.pad(adj_extra.T, ((0, 6), (0, 0)))

    parts = []
    e0 = 0
    for ne in SLABS:
        GAB = _gather(A, B, row, col, e0, ne)
        s = _edge_mlp(GAB, ea_t[:, e0:e0 + ne], d_t[:, e0:e0 + ne],
                      time_emb[e0:e0 + ne], adj_t[:, e0:e0 + ne],
                      Wa, Wd, Wts, Wtc, W1b, small)
        parts.append(_scatter(pos_flat, row, col, s.reshape(ne), e0, ne))
        e0 += ne

    pos_pad2 = jnp.pad(pos_flat, (0, ACC_LEN - 3 * N)).reshape(240, 128)
    out = _combine(parts[0].reshape(NW, 240, 128),
                   parts[1].reshape(NW, 240, 128), pos_pad2)
    return out.reshape(ACC_LEN)[:3 * N].reshape(N, 3)
